# Initial kernel scaffold; baseline (speedup 1.0000x reference)
import jax
import jax.numpy as jnp
from jax.experimental import pallas as pl


def kernel(x, wd, bd, gd, betad, w1, b1, g1, beta1, w2, b2, g2, beta2):
    raise NotImplementedError("write your pallas kernel here")


# speedup vs baseline: 4.3577x; 4.3577x over previous
import jax
import jax.numpy as jnp
from jax.experimental import pallas as pl
from jax.experimental.pallas import tpu as pltpu

EPS = 1e-5
NEG_SLOPE = 0.01
VMEM_LIMIT = 64 * 1024 * 1024


def _mm_stats_kernel(x_ref, w_ref, y_ref, s_ref, ss_ref):
    j = pl.program_id(1)

    @pl.when(j == 0)
    def _():
        s_ref[...] = jnp.zeros_like(s_ref)
        ss_ref[...] = jnp.zeros_like(ss_ref)

    y = jnp.dot(x_ref[...], w_ref[...], preferred_element_type=jnp.float32)
    y_ref[...] = y
    s_ref[0] += jnp.sum(y, axis=0, keepdims=True)
    ss_ref[0] += jnp.sum(y * y, axis=0, keepdims=True)


def _make_conv_kernel(nb, ld, l_valid, k, with_act):
    def body(h_ref, w_ref, *rest):
        if with_act:
            a_ref, b_ref, y_ref, s_ref, ss_ref = rest
        else:
            y_ref, s_ref, ss_ref = rest
        j = pl.program_id(1)

        @pl.when(j == 0)
        def _():
            s_ref[...] = jnp.zeros_like(s_ref)
            ss_ref[...] = jnp.zeros_like(ss_ref)

        for b in range(nb):
            h = h_ref[b]
            if with_act:
                h = h * a_ref[...] + b_ref[...]
                h = jnp.where(h > 0, h, NEG_SLOPE * h)
            zrow = jnp.zeros((1, h.shape[1]), jnp.float32)
            acc = jnp.dot(h.astype(jnp.bfloat16), w_ref[0],
                          preferred_element_type=jnp.float32)
            for dk in range(1, k):
                hs = jnp.concatenate([h[dk:]] + [zrow] * dk, axis=0)
                acc = acc + jnp.dot(hs.astype(jnp.bfloat16), w_ref[dk],
                                    preferred_element_type=jnp.float32)
            y_ref[b] = acc
            tail = acc[l_valid:ld]
            s_ref[0] += (jnp.sum(acc, axis=0, keepdims=True)
                         - jnp.sum(tail, axis=0, keepdims=True))
            ss_ref[0] += (jnp.sum(acc * acc, axis=0, keepdims=True)
                          - jnp.sum(tail * tail, axis=0, keepdims=True))

    return body


def _make_epilogue_kernel(nb, crop):
    def body(o2_ref, res_ref, a2_ref, b2_ref, ad_ref, bd_ref, out_ref):
        for b in range(nb):
            h = res_ref[b]
            zrow = jnp.zeros((1, h.shape[1]), jnp.float32)
            r = jnp.concatenate([h[crop:]] + [zrow] * crop, axis=0)
            y = (o2_ref[b] * a2_ref[...] + b2_ref[...]
                 + r * ad_ref[...] + bd_ref[...])
            out_ref[b] = jnp.where(y > 0, y, NEG_SLOPE * y)

    return body


def _bn_affine(s2, ss2, count, gamma, beta):
    s = jnp.sum(s2, axis=(0, 1))
    ss = jnp.sum(ss2, axis=(0, 1))
    mean = s / count
    var = jnp.maximum(ss / count - mean * mean, 0.0)
    a = gamma * jax.lax.rsqrt(var + EPS)
    b = beta - mean * a
    return a, b


def _stats_specs(F):
    out_shape = (jax.ShapeDtypeStruct((2, 1, F), jnp.float32),
                 jax.ShapeDtypeStruct((2, 1, F), jnp.float32))
    out_specs = (pl.BlockSpec((1, 1, F), lambda i, j: (i, 0, 0)),
                 pl.BlockSpec((1, 1, F), lambda i, j: (i, 0, 0)))
    return out_shape, out_specs


def _cparams():
    return pltpu.CompilerParams(
        dimension_semantics=("parallel", "arbitrary"),
        vmem_limit_bytes=VMEM_LIMIT)


def kernel(x, wd, bd, gd, betad, w1, b1, g1, beta1, w2, b2, g2, beta2):
    N, Cin, L = x.shape
    F = wd.shape[0]
    pool = wd.shape[2]
    k = w1.shape[2]
    Ld = L // pool
    L1 = Ld - (k - 1)
    L2 = L1 - (k - 1)
    Kd = Cin * pool
    f32, bf16 = jnp.float32, jnp.bfloat16

    xcol = (x[:, :, :Ld * pool].reshape(N, Cin, Ld, pool)
            .transpose(0, 2, 1, 3).reshape(N * Ld, Kd).astype(bf16))
    wdm = wd.transpose(1, 2, 0).reshape(Kd, F).astype(bf16)

    stats_shape, stats_specs = _stats_specs(F)

    TM = min(4096, (N * Ld) // 2)
    J1 = (N * Ld) // TM // 2
    h2d, s_h2, ss_h2 = pl.pallas_call(
        _mm_stats_kernel,
        grid=(2, J1),
        in_specs=[pl.BlockSpec((TM, Kd), lambda i, j: (i * J1 + j, 0)),
                  pl.BlockSpec((Kd, F), lambda i, j: (0, 0))],
        out_specs=(pl.BlockSpec((TM, F), lambda i, j: (i * J1 + j, 0)),)
        + stats_specs,
        out_shape=(jax.ShapeDtypeStruct((N * Ld, F), f32),) + stats_shape,
        compiler_params=_cparams(),
    )(xcol, wdm)
    a_d, b_d = _bn_affine(s_h2, ss_h2, N * Ld, gd, betad)
    h3 = h2d.reshape(N, Ld, F)

    NB = 2
    J2 = N // NB // 2
    conv_in = [pl.BlockSpec((NB, Ld, F), lambda i, j: (i * J2 + j, 0, 0)),
               pl.BlockSpec((k, F, F), lambda i, j: (0, 0, 0))]
    row_spec = pl.BlockSpec((1, F), lambda i, j: (0, 0))
    conv_out = (pl.BlockSpec((NB, Ld, F), lambda i, j: (i * J2 + j, 0, 0)),) \
        + stats_specs
    conv_shape = (jax.ShapeDtypeStruct((N, Ld, F), f32),) + stats_shape

    w1f = (w1.transpose(2, 1, 0) * a_d[None, :, None]).astype(bf16)
    o1, s_12, ss_12 = pl.pallas_call(
        _make_conv_kernel(NB, Ld, L1, k, with_act=False),
        grid=(2, J2),
        in_specs=conv_in,
        out_specs=conv_out,
        out_shape=conv_shape,
        compiler_params=_cparams(),
    )(h3, w1f)
    a_1, b_1 = _bn_affine(s_12, ss_12, N * L1, g1, beta1)

    w2f = w2.transpose(2, 1, 0).astype(bf16)
    o2, s_22, ss_22 = pl.pallas_call(
        _make_conv_kernel(NB, Ld, L2, k, with_act=True),
        grid=(2, J2),
        in_specs=conv_in + [row_spec, row_spec],
        out_specs=conv_out,
        out_shape=conv_shape,
        compiler_params=_cparams(),
    )(o1, w2f, a_1.reshape(1, F), b_1.reshape(1, F))
    a_2, b_2 = _bn_affine(s_22, ss_22, N * L2, g2, beta2)

    crop = (Ld - L2) // 2
    row = lambda v: v.reshape(1, F)
    out3 = pl.pallas_call(
        _make_epilogue_kernel(NB, crop),
        grid=(2, J2),
        in_specs=[pl.BlockSpec((NB, Ld, F), lambda i, j: (i * J2 + j, 0, 0)),
                  pl.BlockSpec((NB, Ld, F), lambda i, j: (i * J2 + j, 0, 0)),
                  row_spec, row_spec, row_spec, row_spec],
        out_specs=pl.BlockSpec((NB, Ld, F), lambda i, j: (i * J2 + j, 0, 0)),
        out_shape=jax.ShapeDtypeStruct((N, Ld, F), f32),
        compiler_params=_cparams(),
    )(o2, h3, row(a_2), row(b_2), row(a_d), row(b_d))

    return out3[:, :L2, :].transpose(0, 2, 1)


# speedup vs baseline: 4.6930x; 1.0769x over previous
import jax
import jax.numpy as jnp
from jax.experimental import pallas as pl
from jax.experimental.pallas import tpu as pltpu

EPS = 1e-5
NEG_SLOPE = 0.01
VMEM_LIMIT = 64 * 1024 * 1024


def _mm_stats_kernel(x_ref, w_ref, y_ref, s_ref, ss_ref):
    j = pl.program_id(1)

    @pl.when(j == 0)
    def _():
        s_ref[...] = jnp.zeros_like(s_ref)
        ss_ref[...] = jnp.zeros_like(ss_ref)

    y = jnp.dot(x_ref[...], w_ref[...], preferred_element_type=jnp.float32)
    y_ref[...] = y
    s_ref[0] += jnp.sum(y, axis=0, keepdims=True)
    ss_ref[0] += jnp.sum(y * y, axis=0, keepdims=True)


def _make_conv_kernel(nb, ld, l_valid, k, with_act):
    def body(h_ref, w_ref, *rest):
        if with_act:
            a_ref, b_ref, y_ref, s_ref, ss_ref = rest
        else:
            y_ref, s_ref, ss_ref = rest
        j = pl.program_id(1)

        @pl.when(j == 0)
        def _():
            s_ref[...] = jnp.zeros_like(s_ref)
            ss_ref[...] = jnp.zeros_like(ss_ref)

        for b in range(nb):
            h = h_ref[b]
            if with_act:
                h = h * a_ref[...] + b_ref[...]
                h = jnp.where(h > 0, h, NEG_SLOPE * h)
            zrow = jnp.zeros((1, h.shape[1]), jnp.float32)
            acc = jnp.dot(h.astype(jnp.bfloat16), w_ref[0],
                          preferred_element_type=jnp.float32)
            for dk in range(1, k):
                hs = jnp.concatenate([h[dk:]] + [zrow] * dk, axis=0)
                acc = acc + jnp.dot(hs.astype(jnp.bfloat16), w_ref[dk],
                                    preferred_element_type=jnp.float32)
            y_ref[b] = acc
            tail = acc[l_valid:ld]
            s_ref[0] += (jnp.sum(acc, axis=0, keepdims=True)
                         - jnp.sum(tail, axis=0, keepdims=True))
            ss_ref[0] += (jnp.sum(acc * acc, axis=0, keepdims=True)
                          - jnp.sum(tail * tail, axis=0, keepdims=True))

    return body


def _make_epilogue_kernel(nb, crop, l2):
    def body(o2_ref, res_ref, a2_ref, b2_ref, ad_ref, bd_ref, out_ref):
        for b in range(nb):
            h = res_ref[b]
            zrow = jnp.zeros((1, h.shape[1]), jnp.float32)
            r = jnp.concatenate([h[crop:]] + [zrow] * crop, axis=0)
            y = (o2_ref[b] * a2_ref[...] + b2_ref[...]
                 + r * ad_ref[...] + bd_ref[...])
            y = jnp.where(y > 0, y, NEG_SLOPE * y)
            out_ref[b] = jnp.transpose(y)[:, :l2]

    return body


def _bn_affine(s2, ss2, count, gamma, beta):
    s = jnp.sum(s2, axis=(0, 1))
    ss = jnp.sum(ss2, axis=(0, 1))
    mean = s / count
    var = jnp.maximum(ss / count - mean * mean, 0.0)
    a = gamma * jax.lax.rsqrt(var + EPS)
    b = beta - mean * a
    return a, b


def _stats_specs(F):
    out_shape = (jax.ShapeDtypeStruct((2, 1, F), jnp.float32),
                 jax.ShapeDtypeStruct((2, 1, F), jnp.float32))
    out_specs = (pl.BlockSpec((1, 1, F), lambda i, j: (i, 0, 0)),
                 pl.BlockSpec((1, 1, F), lambda i, j: (i, 0, 0)))
    return out_shape, out_specs


def _cparams():
    return pltpu.CompilerParams(
        dimension_semantics=("parallel", "arbitrary"),
        vmem_limit_bytes=VMEM_LIMIT)


def kernel(x, wd, bd, gd, betad, w1, b1, g1, beta1, w2, b2, g2, beta2):
    N, Cin, L = x.shape
    F = wd.shape[0]
    pool = wd.shape[2]
    k = w1.shape[2]
    Ld = L // pool
    L1 = Ld - (k - 1)
    L2 = L1 - (k - 1)
    Kd = Cin * pool
    f32, bf16 = jnp.float32, jnp.bfloat16

    xcol = (x[:, :, :Ld * pool].reshape(N, Cin, Ld, pool)
            .transpose(0, 2, 1, 3).reshape(N * Ld, Kd).astype(bf16))
    wdm = wd.transpose(1, 2, 0).reshape(Kd, F).astype(bf16)

    stats_shape, stats_specs = _stats_specs(F)

    TM = min(4096, (N * Ld) // 2)
    J1 = (N * Ld) // TM // 2
    h2d, s_h2, ss_h2 = pl.pallas_call(
        _mm_stats_kernel,
        grid=(2, J1),
        in_specs=[pl.BlockSpec((TM, Kd), lambda i, j: (i * J1 + j, 0)),
                  pl.BlockSpec((Kd, F), lambda i, j: (0, 0))],
        out_specs=(pl.BlockSpec((TM, F), lambda i, j: (i * J1 + j, 0)),)
        + stats_specs,
        out_shape=(jax.ShapeDtypeStruct((N * Ld, F), f32),) + stats_shape,
        compiler_params=_cparams(),
    )(xcol, wdm)
    a_d, b_d = _bn_affine(s_h2, ss_h2, N * Ld, gd, betad)
    h3 = h2d.reshape(N, Ld, F)

    NB = 2
    J2 = N // NB // 2
    conv_in = [pl.BlockSpec((NB, Ld, F), lambda i, j: (i * J2 + j, 0, 0)),
               pl.BlockSpec((k, F, F), lambda i, j: (0, 0, 0))]
    row_spec = pl.BlockSpec((1, F), lambda i, j: (0, 0))
    conv_out = (pl.BlockSpec((NB, Ld, F), lambda i, j: (i * J2 + j, 0, 0)),) \
        + stats_specs
    conv_shape = (jax.ShapeDtypeStruct((N, Ld, F), f32),) + stats_shape

    w1f = (w1.transpose(2, 1, 0) * a_d[None, :, None]).astype(bf16)
    o1, s_12, ss_12 = pl.pallas_call(
        _make_conv_kernel(NB, Ld, L1, k, with_act=False),
        grid=(2, J2),
        in_specs=conv_in,
        out_specs=conv_out,
        out_shape=conv_shape,
        compiler_params=_cparams(),
    )(h3, w1f)
    a_1, b_1 = _bn_affine(s_12, ss_12, N * L1, g1, beta1)

    w2f = w2.transpose(2, 1, 0).astype(bf16)
    o2, s_22, ss_22 = pl.pallas_call(
        _make_conv_kernel(NB, Ld, L2, k, with_act=True),
        grid=(2, J2),
        in_specs=conv_in + [row_spec, row_spec],
        out_specs=conv_out,
        out_shape=conv_shape,
        compiler_params=_cparams(),
    )(o1, w2f, a_1.reshape(1, F), b_1.reshape(1, F))
    a_2, b_2 = _bn_affine(s_22, ss_22, N * L2, g2, beta2)

    crop = (Ld - L2) // 2
    row = lambda v: v.reshape(1, F)
    return pl.pallas_call(
        _make_epilogue_kernel(NB, crop, L2),
        grid=(2, J2),
        in_specs=[pl.BlockSpec((NB, Ld, F), lambda i, j: (i * J2 + j, 0, 0)),
                  pl.BlockSpec((NB, Ld, F), lambda i, j: (i * J2 + j, 0, 0)),
                  row_spec, row_spec, row_spec, row_spec],
        out_specs=pl.BlockSpec((NB, F, L2), lambda i, j: (i * J2 + j, 0, 0)),
        out_shape=jax.ShapeDtypeStruct((N, F, L2), f32),
        compiler_params=_cparams(),
    )(o2, h3, row(a_2), row(b_2), row(a_d), row(b_d))


# speedup vs baseline: 6.0848x; 1.2966x over previous
import jax
import jax.numpy as jnp
from jax.experimental import pallas as pl
from jax.experimental.pallas import tpu as pltpu

EPS = 1e-5
NEG_SLOPE = 0.01
VMEM_LIMIT = 64 * 1024 * 1024


def _make_expand_kernel(nb, ld, pool):
    def body(x_ref, w_ref, y_ref, s_ref, ss_ref):
        j = pl.program_id(1)

        @pl.when(j == 0)
        def _():
            s_ref[...] = jnp.zeros_like(s_ref)
            ss_ref[...] = jnp.zeros_like(ss_ref)

        for b in range(nb):
            x2 = x_ref[b]
            t = jnp.transpose(x2)
            t2 = t.reshape(ld, pool, x2.shape[0])
            xcol = jnp.concatenate([t2[:, p, :] for p in range(pool)], axis=1)
            y = jnp.dot(xcol.astype(jnp.bfloat16), w_ref[...],
                        preferred_element_type=jnp.float32)
            y_ref[b] = y
            s_ref[0] += jnp.sum(y, axis=0, keepdims=True)
            ss_ref[0] += jnp.sum(y * y, axis=0, keepdims=True)

    return body


def _make_conv_kernel(nb, ld, l_valid, k, with_act):
    def body(h_ref, w_ref, *rest):
        if with_act:
            a_ref, b_ref, y_ref, s_ref, ss_ref = rest
        else:
            y_ref, s_ref, ss_ref = rest
        j = pl.program_id(1)

        @pl.when(j == 0)
        def _():
            s_ref[...] = jnp.zeros_like(s_ref)
            ss_ref[...] = jnp.zeros_like(ss_ref)

        for b in range(nb):
            h = h_ref[b]
            if with_act:
                h = h * a_ref[...] + b_ref[...]
                h = jnp.where(h > 0, h, NEG_SLOPE * h)
            zrow = jnp.zeros((1, h.shape[1]), jnp.float32)
            acc = jnp.dot(h.astype(jnp.bfloat16), w_ref[0],
                          preferred_element_type=jnp.float32)
            for dk in range(1, k):
                hs = jnp.concatenate([h[dk:]] + [zrow] * dk, axis=0)
                acc = acc + jnp.dot(hs.astype(jnp.bfloat16), w_ref[dk],
                                    preferred_element_type=jnp.float32)
            y_ref[b] = acc
            tail = acc[l_valid:ld]
            s_ref[0] += (jnp.sum(acc, axis=0, keepdims=True)
                         - jnp.sum(tail, axis=0, keepdims=True))
            ss_ref[0] += (jnp.sum(acc * acc, axis=0, keepdims=True)
                          - jnp.sum(tail * tail, axis=0, keepdims=True))

    return body


def _make_epilogue_kernel(nb, crop, l2):
    def body(o2_ref, res_ref, a2_ref, b2_ref, ad_ref, bd_ref, out_ref):
        for b in range(nb):
            h = res_ref[b]
            zrow = jnp.zeros((1, h.shape[1]), jnp.float32)
            r = jnp.concatenate([h[crop:]] + [zrow] * crop, axis=0)
            y = (o2_ref[b] * a2_ref[...] + b2_ref[...]
                 + r * ad_ref[...] + bd_ref[...])
            y = jnp.where(y > 0, y, NEG_SLOPE * y)
            out_ref[b] = jnp.transpose(y)[:, :l2]

    return body


def _bn_affine(s2, ss2, count, gamma, beta):
    s = jnp.sum(s2, axis=(0, 1))
    ss = jnp.sum(ss2, axis=(0, 1))
    mean = s / count
    var = jnp.maximum(ss / count - mean * mean, 0.0)
    a = gamma * jax.lax.rsqrt(var + EPS)
    b = beta - mean * a
    return a, b


def _stats_specs(F):
    out_shape = (jax.ShapeDtypeStruct((2, 1, F), jnp.float32),
                 jax.ShapeDtypeStruct((2, 1, F), jnp.float32))
    out_specs = (pl.BlockSpec((1, 1, F), lambda i, j: (i, 0, 0)),
                 pl.BlockSpec((1, 1, F), lambda i, j: (i, 0, 0)))
    return out_shape, out_specs


def _cparams():
    return pltpu.CompilerParams(
        dimension_semantics=("parallel", "arbitrary"),
        vmem_limit_bytes=VMEM_LIMIT)


def kernel(x, wd, bd, gd, betad, w1, b1, g1, beta1, w2, b2, g2, beta2):
    N, Cin, L = x.shape
    F = wd.shape[0]
    pool = wd.shape[2]
    k = w1.shape[2]
    Ld = L // pool
    L1 = Ld - (k - 1)
    L2 = L1 - (k - 1)
    Kd = Cin * pool
    f32, bf16 = jnp.float32, jnp.bfloat16

    wdm = wd.transpose(2, 1, 0).reshape(Kd, F).astype(bf16)

    stats_shape, stats_specs = _stats_specs(F)
    NB = 2
    J2 = N // NB // 2

    h3, s_h2, ss_h2 = pl.pallas_call(
        _make_expand_kernel(NB, Ld, pool),
        grid=(2, J2),
        in_specs=[pl.BlockSpec((NB, Cin, L), lambda i, j: (i * J2 + j, 0, 0)),
                  pl.BlockSpec((Kd, F), lambda i, j: (0, 0))],
        out_specs=(pl.BlockSpec((NB, Ld, F), lambda i, j: (i * J2 + j, 0, 0)),)
        + stats_specs,
        out_shape=(jax.ShapeDtypeStruct((N, Ld, F), f32),) + stats_shape,
        compiler_params=_cparams(),
    )(x, wdm)
    a_d, b_d = _bn_affine(s_h2, ss_h2, N * Ld, gd, betad)
    conv_in = [pl.BlockSpec((NB, Ld, F), lambda i, j: (i * J2 + j, 0, 0)),
               pl.BlockSpec((k, F, F), lambda i, j: (0, 0, 0))]
    row_spec = pl.BlockSpec((1, F), lambda i, j: (0, 0))
    conv_out = (pl.BlockSpec((NB, Ld, F), lambda i, j: (i * J2 + j, 0, 0)),) \
        + stats_specs
    conv_shape = (jax.ShapeDtypeStruct((N, Ld, F), f32),) + stats_shape

    w1f = (w1.transpose(2, 1, 0) * a_d[None, :, None]).astype(bf16)
    o1, s_12, ss_12 = pl.pallas_call(
        _make_conv_kernel(NB, Ld, L1, k, with_act=False),
        grid=(2, J2),
        in_specs=conv_in,
        out_specs=conv_out,
        out_shape=conv_shape,
        compiler_params=_cparams(),
    )(h3, w1f)
    a_1, b_1 = _bn_affine(s_12, ss_12, N * L1, g1, beta1)

    w2f = w2.transpose(2, 1, 0).astype(bf16)
    o2, s_22, ss_22 = pl.pallas_call(
        _make_conv_kernel(NB, Ld, L2, k, with_act=True),
        grid=(2, J2),
        in_specs=conv_in + [row_spec, row_spec],
        out_specs=conv_out,
        out_shape=conv_shape,
        compiler_params=_cparams(),
    )(o1, w2f, a_1.reshape(1, F), b_1.reshape(1, F))
    a_2, b_2 = _bn_affine(s_22, ss_22, N * L2, g2, beta2)

    crop = (Ld - L2) // 2
    row = lambda v: v.reshape(1, F)
    return pl.pallas_call(
        _make_epilogue_kernel(NB, crop, L2),
        grid=(2, J2),
        in_specs=[pl.BlockSpec((NB, Ld, F), lambda i, j: (i * J2 + j, 0, 0)),
                  pl.BlockSpec((NB, Ld, F), lambda i, j: (i * J2 + j, 0, 0)),
                  row_spec, row_spec, row_spec, row_spec],
        out_specs=pl.BlockSpec((NB, F, L2), lambda i, j: (i * J2 + j, 0, 0)),
        out_shape=jax.ShapeDtypeStruct((N, F, L2), f32),
        compiler_params=_cparams(),
    )(o2, h3, row(a_2), row(b_2), row(a_d), row(b_d))


# speedup vs baseline: 6.7258x; 1.1053x over previous
import jax
import jax.numpy as jnp
from jax.experimental import pallas as pl
from jax.experimental.pallas import tpu as pltpu

EPS = 1e-5
NEG_SLOPE = 0.01
VMEM_LIMIT = 64 * 1024 * 1024


def _make_expand_kernel(nb, ld, pool):
    def body(x_ref, w_ref, y_ref, s_ref, ss_ref):
        j = pl.program_id(1)

        @pl.when(j == 0)
        def _():
            s_ref[...] = jnp.zeros_like(s_ref)
            ss_ref[...] = jnp.zeros_like(ss_ref)

        for b in range(nb):
            x2 = x_ref[b]
            t = jnp.transpose(x2)
            t2 = t.reshape(ld, pool, x2.shape[0])
            xcol = jnp.concatenate([t2[:, p, :] for p in range(pool)], axis=1)
            y = jnp.dot(xcol.astype(jnp.bfloat16), w_ref[...],
                        preferred_element_type=jnp.float32)
            y_ref[b] = y.astype(jnp.bfloat16)
            s_ref[0] += jnp.sum(y, axis=0, keepdims=True)
            ss_ref[0] += jnp.sum(y * y, axis=0, keepdims=True)

    return body


def _make_conv_kernel(nb, ld, l_valid, k, with_act):
    def body(h_ref, w_ref, *rest):
        if with_act:
            a_ref, b_ref, y_ref, s_ref, ss_ref = rest
        else:
            y_ref, s_ref, ss_ref = rest
        j = pl.program_id(1)

        @pl.when(j == 0)
        def _():
            s_ref[...] = jnp.zeros_like(s_ref)
            ss_ref[...] = jnp.zeros_like(ss_ref)

        for b in range(nb):
            h = h_ref[b]
            if with_act:
                hf = h.astype(jnp.float32)
                hf = hf * a_ref[...] + b_ref[...]
                hf = jnp.where(hf > 0, hf, NEG_SLOPE * hf)
                h = hf.astype(jnp.bfloat16)
            zrow = jnp.zeros((1, h.shape[1]), jnp.bfloat16)
            acc = jnp.dot(h, w_ref[0], preferred_element_type=jnp.float32)
            for dk in range(1, k):
                hs = jnp.concatenate([h[dk:]] + [zrow] * dk, axis=0)
                acc = acc + jnp.dot(hs, w_ref[dk],
                                    preferred_element_type=jnp.float32)
            y_ref[b] = acc.astype(jnp.bfloat16)
            tail = acc[l_valid:ld]
            s_ref[0] += (jnp.sum(acc, axis=0, keepdims=True)
                         - jnp.sum(tail, axis=0, keepdims=True))
            ss_ref[0] += (jnp.sum(acc * acc, axis=0, keepdims=True)
                          - jnp.sum(tail * tail, axis=0, keepdims=True))

    return body


def _make_epilogue_kernel(nb, crop, l2):
    def body(o2_ref, res_ref, a2_ref, b2_ref, ad_ref, bd_ref, out_ref):
        for b in range(nb):
            h = res_ref[b].astype(jnp.float32)
            zrow = jnp.zeros((1, h.shape[1]), jnp.float32)
            r = jnp.concatenate([h[crop:]] + [zrow] * crop, axis=0)
            y = (o2_ref[b].astype(jnp.float32) * a2_ref[...] + b2_ref[...]
                 + r * ad_ref[...] + bd_ref[...])
            y = jnp.where(y > 0, y, NEG_SLOPE * y)
            out_ref[b] = jnp.transpose(y)[:, :l2]

    return body


def _bn_affine(s2, ss2, count, gamma, beta):
    s = jnp.sum(s2, axis=(0, 1))
    ss = jnp.sum(ss2, axis=(0, 1))
    mean = s / count
    var = jnp.maximum(ss / count - mean * mean, 0.0)
    a = gamma * jax.lax.rsqrt(var + EPS)
    b = beta - mean * a
    return a, b


def _stats_specs(F):
    out_shape = (jax.ShapeDtypeStruct((2, 1, F), jnp.float32),
                 jax.ShapeDtypeStruct((2, 1, F), jnp.float32))
    out_specs = (pl.BlockSpec((1, 1, F), lambda i, j: (i, 0, 0)),
                 pl.BlockSpec((1, 1, F), lambda i, j: (i, 0, 0)))
    return out_shape, out_specs


def _cparams():
    return pltpu.CompilerParams(
        dimension_semantics=("parallel", "arbitrary"),
        vmem_limit_bytes=VMEM_LIMIT)


def kernel(x, wd, bd, gd, betad, w1, b1, g1, beta1, w2, b2, g2, beta2):
    N, Cin, L = x.shape
    F = wd.shape[0]
    pool = wd.shape[2]
    k = w1.shape[2]
    Ld = L // pool
    L1 = Ld - (k - 1)
    L2 = L1 - (k - 1)
    Kd = Cin * pool
    f32, bf16 = jnp.float32, jnp.bfloat16

    wdm = wd.transpose(2, 1, 0).reshape(Kd, F).astype(bf16)

    stats_shape, stats_specs = _stats_specs(F)
    NB = 2
    J2 = N // NB // 2

    h3, s_h2, ss_h2 = pl.pallas_call(
        _make_expand_kernel(NB, Ld, pool),
        grid=(2, J2),
        in_specs=[pl.BlockSpec((NB, Cin, L), lambda i, j: (i * J2 + j, 0, 0)),
                  pl.BlockSpec((Kd, F), lambda i, j: (0, 0))],
        out_specs=(pl.BlockSpec((NB, Ld, F), lambda i, j: (i * J2 + j, 0, 0)),)
        + stats_specs,
        out_shape=(jax.ShapeDtypeStruct((N, Ld, F), bf16),) + stats_shape,
        compiler_params=_cparams(),
    )(x, wdm)
    a_d, b_d = _bn_affine(s_h2, ss_h2, N * Ld, gd, betad)
    conv_in = [pl.BlockSpec((NB, Ld, F), lambda i, j: (i * J2 + j, 0, 0)),
               pl.BlockSpec((k, F, F), lambda i, j: (0, 0, 0))]
    row_spec = pl.BlockSpec((1, F), lambda i, j: (0, 0))
    conv_out = (pl.BlockSpec((NB, Ld, F), lambda i, j: (i * J2 + j, 0, 0)),) \
        + stats_specs
    conv_shape = (jax.ShapeDtypeStruct((N, Ld, F), bf16),) + stats_shape

    w1f = (w1.transpose(2, 1, 0) * a_d[None, :, None]).astype(bf16)
    o1, s_12, ss_12 = pl.pallas_call(
        _make_conv_kernel(NB, Ld, L1, k, with_act=False),
        grid=(2, J2),
        in_specs=conv_in,
        out_specs=conv_out,
        out_shape=conv_shape,
        compiler_params=_cparams(),
    )(h3, w1f)
    a_1, b_1 = _bn_affine(s_12, ss_12, N * L1, g1, beta1)

    w2f = w2.transpose(2, 1, 0).astype(bf16)
    o2, s_22, ss_22 = pl.pallas_call(
        _make_conv_kernel(NB, Ld, L2, k, with_act=True),
        grid=(2, J2),
        in_specs=conv_in + [row_spec, row_spec],
        out_specs=conv_out,
        out_shape=conv_shape,
        compiler_params=_cparams(),
    )(o1, w2f, a_1.reshape(1, F), b_1.reshape(1, F))
    a_2, b_2 = _bn_affine(s_22, ss_22, N * L2, g2, beta2)

    crop = (Ld - L2) // 2
    row = lambda v: v.reshape(1, F)
    return pl.pallas_call(
        _make_epilogue_kernel(NB, crop, L2),
        grid=(2, J2),
        in_specs=[pl.BlockSpec((NB, Ld, F), lambda i, j: (i * J2 + j, 0, 0)),
                  pl.BlockSpec((NB, Ld, F), lambda i, j: (i * J2 + j, 0, 0)),
                  row_spec, row_spec, row_spec, row_spec],
        out_specs=pl.BlockSpec((NB, F, L2), lambda i, j: (i * J2 + j, 0, 0)),
        out_shape=jax.ShapeDtypeStruct((N, F, L2), f32),
        compiler_params=_cparams(),
    )(o2, h3, row(a_2), row(b_2), row(a_d), row(b_d))


# speedup vs baseline: 7.4973x; 1.1147x over previous
import jax
import jax.numpy as jnp
from jax.experimental import pallas as pl
from jax.experimental.pallas import tpu as pltpu

EPS = 1e-5
NEG_SLOPE = 0.01
VMEM_LIMIT = 64 * 1024 * 1024


def _make_expand_kernel(nb, ld, pool):
    def body(x_ref, w_ref, y_ref, s_ref, ss_ref):
        j = pl.program_id(1)

        @pl.when(j == 0)
        def _():
            s_ref[...] = jnp.zeros_like(s_ref)
            ss_ref[...] = jnp.zeros_like(ss_ref)

        for b in range(nb):
            x2 = x_ref[b]
            cin, l = x2.shape
            idx = jnp.concatenate(
                [jnp.arange(p, 128, pool) for p in range(pool)])
            idxb = jnp.broadcast_to(idx[None, :], (cin, 128))
            gs = [jnp.take_along_axis(x2[:, 128 * j:128 * (j + 1)], idxb,
                                      axis=1)
                  for j in range(l // 128)]
            w = 128 // pool
            xps = [jnp.concatenate([g[:, w * p:w * (p + 1)] for g in gs],
                                   axis=1)
                   for p in range(pool)]
            xcat = jnp.concatenate(xps, axis=0)
            y = jax.lax.dot_general(
                xcat.astype(jnp.bfloat16), w_ref[...],
                dimension_numbers=(((0,), (0,)), ((), ())),
                preferred_element_type=jnp.float32)
            y_ref[b] = y.astype(jnp.bfloat16)
            s_ref[0] += jnp.sum(y, axis=0, keepdims=True)
            ss_ref[0] += jnp.sum(y * y, axis=0, keepdims=True)

    return body


def _make_conv_kernel(nb, ld, l_valid, k, with_act):
    def body(h_ref, w_ref, *rest):
        if with_act:
            a_ref, b_ref, y_ref, s_ref, ss_ref = rest
        else:
            y_ref, s_ref, ss_ref = rest
        j = pl.program_id(1)

        @pl.when(j == 0)
        def _():
            s_ref[...] = jnp.zeros_like(s_ref)
            ss_ref[...] = jnp.zeros_like(ss_ref)

        for b in range(nb):
            h = h_ref[b]
            if with_act:
                hf = h.astype(jnp.float32)
                hf = hf * a_ref[...] + b_ref[...]
                hf = jnp.where(hf > 0, hf, NEG_SLOPE * hf)
                h = hf.astype(jnp.bfloat16)
            zrow = jnp.zeros((1, h.shape[1]), jnp.bfloat16)
            acc = jnp.dot(h, w_ref[0], preferred_element_type=jnp.float32)
            for dk in range(1, k):
                hs = jnp.concatenate([h[dk:]] + [zrow] * dk, axis=0)
                acc = acc + jnp.dot(hs, w_ref[dk],
                                    preferred_element_type=jnp.float32)
            y_ref[b] = acc.astype(jnp.bfloat16)
            tail = acc[l_valid:ld]
            s_ref[0] += (jnp.sum(acc, axis=0, keepdims=True)
                         - jnp.sum(tail, axis=0, keepdims=True))
            ss_ref[0] += (jnp.sum(acc * acc, axis=0, keepdims=True)
                          - jnp.sum(tail * tail, axis=0, keepdims=True))

    return body


def _make_epilogue_kernel(nb, crop, l2):
    def body(o2_ref, res_ref, a2_ref, b2_ref, ad_ref, bd_ref, out_ref):
        for b in range(nb):
            h = res_ref[b].astype(jnp.float32)
            zrow = jnp.zeros((1, h.shape[1]), jnp.float32)
            r = jnp.concatenate([h[crop:]] + [zrow] * crop, axis=0)
            y = (o2_ref[b].astype(jnp.float32) * a2_ref[...] + b2_ref[...]
                 + r * ad_ref[...] + bd_ref[...])
            y = jnp.where(y > 0, y, NEG_SLOPE * y)
            out_ref[b] = jnp.transpose(y)

    return body


def _bn_affine(s2, ss2, count, gamma, beta):
    s = jnp.sum(s2, axis=(0, 1))
    ss = jnp.sum(ss2, axis=(0, 1))
    mean = s / count
    var = jnp.maximum(ss / count - mean * mean, 0.0)
    a = gamma * jax.lax.rsqrt(var + EPS)
    b = beta - mean * a
    return a, b


def _stats_specs(F):
    out_shape = (jax.ShapeDtypeStruct((2, 1, F), jnp.float32),
                 jax.ShapeDtypeStruct((2, 1, F), jnp.float32))
    out_specs = (pl.BlockSpec((1, 1, F), lambda i, j: (i, 0, 0)),
                 pl.BlockSpec((1, 1, F), lambda i, j: (i, 0, 0)))
    return out_shape, out_specs


def _cparams():
    return pltpu.CompilerParams(
        dimension_semantics=("parallel", "arbitrary"),
        vmem_limit_bytes=VMEM_LIMIT)


def kernel(x, wd, bd, gd, betad, w1, b1, g1, beta1, w2, b2, g2, beta2):
    N, Cin, L = x.shape
    F = wd.shape[0]
    pool = wd.shape[2]
    k = w1.shape[2]
    Ld = L // pool
    L1 = Ld - (k - 1)
    L2 = L1 - (k - 1)
    Kd = Cin * pool
    f32, bf16 = jnp.float32, jnp.bfloat16

    wdm = wd.transpose(2, 1, 0).reshape(Kd, F).astype(bf16)

    stats_shape, stats_specs = _stats_specs(F)
    NB = 2
    J2 = N // NB // 2

    h3, s_h2, ss_h2 = pl.pallas_call(
        _make_expand_kernel(NB, Ld, pool),
        grid=(2, J2),
        in_specs=[pl.BlockSpec((NB, Cin, L), lambda i, j: (i * J2 + j, 0, 0)),
                  pl.BlockSpec((Kd, F), lambda i, j: (0, 0))],
        out_specs=(pl.BlockSpec((NB, Ld, F), lambda i, j: (i * J2 + j, 0, 0)),)
        + stats_specs,
        out_shape=(jax.ShapeDtypeStruct((N, Ld, F), bf16),) + stats_shape,
        compiler_params=_cparams(),
    )(x, wdm)
    a_d, b_d = _bn_affine(s_h2, ss_h2, N * Ld, gd, betad)
    conv_in = [pl.BlockSpec((NB, Ld, F), lambda i, j: (i * J2 + j, 0, 0)),
               pl.BlockSpec((k, F, F), lambda i, j: (0, 0, 0))]
    row_spec = pl.BlockSpec((1, F), lambda i, j: (0, 0))
    conv_out = (pl.BlockSpec((NB, Ld, F), lambda i, j: (i * J2 + j, 0, 0)),) \
        + stats_specs
    conv_shape = (jax.ShapeDtypeStruct((N, Ld, F), bf16),) + stats_shape

    w1f = (w1.transpose(2, 1, 0) * a_d[None, :, None]).astype(bf16)
    o1, s_12, ss_12 = pl.pallas_call(
        _make_conv_kernel(NB, Ld, L1, k, with_act=False),
        grid=(2, J2),
        in_specs=conv_in,
        out_specs=conv_out,
        out_shape=conv_shape,
        compiler_params=_cparams(),
    )(h3, w1f)
    a_1, b_1 = _bn_affine(s_12, ss_12, N * L1, g1, beta1)

    w2f = w2.transpose(2, 1, 0).astype(bf16)
    o2, s_22, ss_22 = pl.pallas_call(
        _make_conv_kernel(NB, Ld, L2, k, with_act=True),
        grid=(2, J2),
        in_specs=conv_in + [row_spec, row_spec],
        out_specs=conv_out,
        out_shape=conv_shape,
        compiler_params=_cparams(),
    )(o1, w2f, a_1.reshape(1, F), b_1.reshape(1, F))
    a_2, b_2 = _bn_affine(s_22, ss_22, N * L2, g2, beta2)

    crop = (Ld - L2) // 2
    row = lambda v: v.reshape(1, F)
    out = pl.pallas_call(
        _make_epilogue_kernel(NB, crop, L2),
        grid=(2, J2),
        in_specs=[pl.BlockSpec((NB, Ld, F), lambda i, j: (i * J2 + j, 0, 0)),
                  pl.BlockSpec((NB, Ld, F), lambda i, j: (i * J2 + j, 0, 0)),
                  row_spec, row_spec, row_spec, row_spec],
        out_specs=pl.BlockSpec((NB, F, Ld), lambda i, j: (i * J2 + j, 0, 0)),
        out_shape=jax.ShapeDtypeStruct((N, F, Ld), f32),
        compiler_params=_cparams(),
    )(o2, h3, row(a_2), row(b_2), row(a_d), row(b_d))
    return out[:, :, :L2]


# speedup vs baseline: 8.2546x; 1.1010x over previous
import jax
import jax.numpy as jnp
from jax.experimental import pallas as pl
from jax.experimental.pallas import tpu as pltpu

EPS = 1e-5
NEG_SLOPE = 0.01
VMEM_LIMIT = 64 * 1024 * 1024


def _make_expand_kernel(nb, ld, pool):
    def body(x_ref, w_ref, y_ref, s_ref, ss_ref):
        j = pl.program_id(1)

        @pl.when(j == 0)
        def _():
            s_ref[...] = jnp.zeros_like(s_ref)
            ss_ref[...] = jnp.zeros_like(ss_ref)

        for b in range(nb):
            x2 = x_ref[b]
            cin, l = x2.shape
            idx = jnp.concatenate(
                [jnp.arange(p, 128, pool) for p in range(pool)])
            idxb = jnp.broadcast_to(idx[None, :], (cin, 128))
            gs = [jnp.take_along_axis(x2[:, 128 * j:128 * (j + 1)], idxb,
                                      axis=1)
                  for j in range(l // 128)]
            w = 128 // pool
            xps = [jnp.concatenate([g[:, w * p:w * (p + 1)] for g in gs],
                                   axis=1)
                   for p in range(pool)]
            xcat = jnp.concatenate(xps, axis=0)
            y = jax.lax.dot_general(
                xcat.astype(jnp.bfloat16), w_ref[...],
                dimension_numbers=(((0,), (0,)), ((), ())),
                preferred_element_type=jnp.float32)
            y_ref[b] = y.astype(jnp.bfloat16)
            s_ref[0] += jnp.sum(y, axis=0, keepdims=True)
            ss_ref[0] += jnp.sum(y * y, axis=0, keepdims=True)

    return body


def _make_conv_kernel(nb, ld, l_valid, k, with_act):
    def body(h_ref, w_ref, *rest):
        if with_act:
            a_ref, b_ref, y_ref, s_ref, ss_ref = rest
        else:
            y_ref, s_ref, ss_ref = rest
        j = pl.program_id(1)

        @pl.when(j == 0)
        def _():
            s_ref[...] = jnp.zeros_like(s_ref)
            ss_ref[...] = jnp.zeros_like(ss_ref)

        for b in range(nb):
            h = h_ref[b]
            if with_act:
                hf = h.astype(jnp.float32)
                hf = hf * a_ref[...] + b_ref[...]
                hf = jnp.where(hf > 0, hf, NEG_SLOPE * hf)
                h = hf.astype(jnp.bfloat16)
            zrow = jnp.zeros((1, h.shape[1]), jnp.bfloat16)
            acc = jnp.dot(h, w_ref[0], preferred_element_type=jnp.float32)
            for dk in range(1, k):
                hs = jnp.concatenate([h[dk:]] + [zrow] * dk, axis=0)
                acc = acc + jnp.dot(hs, w_ref[dk],
                                    preferred_element_type=jnp.float32)
            y_ref[b] = acc.astype(jnp.bfloat16)
            tail = acc[l_valid:ld]
            s_ref[0] += (jnp.sum(acc, axis=0, keepdims=True)
                         - jnp.sum(tail, axis=0, keepdims=True))
            ss_ref[0] += (jnp.sum(acc * acc, axis=0, keepdims=True)
                          - jnp.sum(tail * tail, axis=0, keepdims=True))

    return body


def _make_epilogue_kernel(nb, crop, l2):
    def body(o2_ref, res_ref, a2_ref, b2_ref, ad_ref, bd_ref, out_ref):
        for b in range(nb):
            h = res_ref[b].astype(jnp.float32)
            zrow = jnp.zeros((1, h.shape[1]), jnp.float32)
            r = jnp.concatenate([h[crop:]] + [zrow] * crop, axis=0)
            y = (o2_ref[b].astype(jnp.float32) * a2_ref[...] + b2_ref[...]
                 + r * ad_ref[...] + bd_ref[...])
            y = jnp.where(y > 0, y, NEG_SLOPE * y)
            out_ref[b] = jnp.transpose(y)

    return body


def _bn_affine(s2, ss2, count, gamma, beta):
    s = jnp.sum(s2, axis=(0, 1))
    ss = jnp.sum(ss2, axis=(0, 1))
    mean = s / count
    var = jnp.maximum(ss / count - mean * mean, 0.0)
    a = gamma * jax.lax.rsqrt(var + EPS)
    b = beta - mean * a
    return a, b


def _stats_specs(F):
    out_shape = (jax.ShapeDtypeStruct((2, 1, F), jnp.float32),
                 jax.ShapeDtypeStruct((2, 1, F), jnp.float32))
    out_specs = (pl.BlockSpec((1, 1, F), lambda i, j: (i, 0, 0)),
                 pl.BlockSpec((1, 1, F), lambda i, j: (i, 0, 0)))
    return out_shape, out_specs


def _cparams():
    return pltpu.CompilerParams(
        dimension_semantics=("parallel", "arbitrary"),
        vmem_limit_bytes=VMEM_LIMIT)


def kernel(x, wd, bd, gd, betad, w1, b1, g1, beta1, w2, b2, g2, beta2):
    N, Cin, L = x.shape
    F = wd.shape[0]
    pool = wd.shape[2]
    k = w1.shape[2]
    Ld = L // pool
    L1 = Ld - (k - 1)
    L2 = L1 - (k - 1)
    Kd = Cin * pool
    f32, bf16 = jnp.float32, jnp.bfloat16

    wdm = wd.transpose(2, 1, 0).reshape(Kd, F).astype(bf16)

    stats_shape, stats_specs = _stats_specs(F)
    NB = 4
    J2 = N // NB // 2

    h3, s_h2, ss_h2 = pl.pallas_call(
        _make_expand_kernel(NB, Ld, pool),
        grid=(2, J2),
        in_specs=[pl.BlockSpec((NB, Cin, L), lambda i, j: (i * J2 + j, 0, 0)),
                  pl.BlockSpec((Kd, F), lambda i, j: (0, 0))],
        out_specs=(pl.BlockSpec((NB, Ld, F), lambda i, j: (i * J2 + j, 0, 0)),)
        + stats_specs,
        out_shape=(jax.ShapeDtypeStruct((N, Ld, F), bf16),) + stats_shape,
        compiler_params=_cparams(),
    )(x, wdm)
    a_d, b_d = _bn_affine(s_h2, ss_h2, N * Ld, gd, betad)
    conv_in = [pl.BlockSpec((NB, Ld, F), lambda i, j: (i * J2 + j, 0, 0)),
               pl.BlockSpec((k, F, F), lambda i, j: (0, 0, 0))]
    row_spec = pl.BlockSpec((1, F), lambda i, j: (0, 0))
    conv_out = (pl.BlockSpec((NB, Ld, F), lambda i, j: (i * J2 + j, 0, 0)),) \
        + stats_specs
    conv_shape = (jax.ShapeDtypeStruct((N, Ld, F), bf16),) + stats_shape

    w1f = (w1.transpose(2, 1, 0) * a_d[None, :, None]).astype(bf16)
    o1, s_12, ss_12 = pl.pallas_call(
        _make_conv_kernel(NB, Ld, L1, k, with_act=False),
        grid=(2, J2),
        in_specs=conv_in,
        out_specs=conv_out,
        out_shape=conv_shape,
        compiler_params=_cparams(),
    )(h3, w1f)
    a_1, b_1 = _bn_affine(s_12, ss_12, N * L1, g1, beta1)

    w2f = w2.transpose(2, 1, 0).astype(bf16)
    o2, s_22, ss_22 = pl.pallas_call(
        _make_conv_kernel(NB, Ld, L2, k, with_act=True),
        grid=(2, J2),
        in_specs=conv_in + [row_spec, row_spec],
        out_specs=conv_out,
        out_shape=conv_shape,
        compiler_params=_cparams(),
    )(o1, w2f, a_1.reshape(1, F), b_1.reshape(1, F))
    a_2, b_2 = _bn_affine(s_22, ss_22, N * L2, g2, beta2)

    crop = (Ld - L2) // 2
    row = lambda v: v.reshape(1, F)
    out = pl.pallas_call(
        _make_epilogue_kernel(NB, crop, L2),
        grid=(2, J2),
        in_specs=[pl.BlockSpec((NB, Ld, F), lambda i, j: (i * J2 + j, 0, 0)),
                  pl.BlockSpec((NB, Ld, F), lambda i, j: (i * J2 + j, 0, 0)),
                  row_spec, row_spec, row_spec, row_spec],
        out_specs=pl.BlockSpec((NB, F, Ld), lambda i, j: (i * J2 + j, 0, 0)),
        out_shape=jax.ShapeDtypeStruct((N, F, Ld), f32),
        compiler_params=_cparams(),
    )(o2, h3, row(a_2), row(b_2), row(a_d), row(b_d))
    return out[:, :, :L2]


# speedup vs baseline: 8.4312x; 1.0214x over previous
import jax
import jax.numpy as jnp
from jax.experimental import pallas as pl
from jax.experimental.pallas import tpu as pltpu

EPS = 1e-5
NEG_SLOPE = 0.01
VMEM_LIMIT = 64 * 1024 * 1024


def _make_expand_kernel(nb, ld, pool):
    def body(x_ref, w_ref, y_ref, s_ref, ss_ref):
        j = pl.program_id(1)

        @pl.when(j == 0)
        def _():
            s_ref[...] = jnp.zeros_like(s_ref)
            ss_ref[...] = jnp.zeros_like(ss_ref)

        for b in range(nb):
            x2 = x_ref[b]
            cin, l = x2.shape
            idx = jnp.concatenate(
                [jnp.arange(p, 128, pool) for p in range(pool)])
            idxb = jnp.broadcast_to(idx[None, :], (cin, 128))
            gs = [jnp.take_along_axis(x2[:, 128 * j:128 * (j + 1)], idxb,
                                      axis=1)
                  for j in range(l // 128)]
            w = 128 // pool
            xps = [jnp.concatenate([g[:, w * p:w * (p + 1)] for g in gs],
                                   axis=1)
                   for p in range(pool)]
            xcat = jnp.concatenate(xps, axis=0)
            y = jax.lax.dot_general(
                xcat.astype(jnp.bfloat16), w_ref[...],
                dimension_numbers=(((0,), (0,)), ((), ())),
                preferred_element_type=jnp.float32)
            y_ref[b] = y.astype(jnp.bfloat16)
            s_ref[0] += jnp.sum(y, axis=0, keepdims=True)
            ss_ref[0] += jnp.sum(y * y, axis=0, keepdims=True)

    return body


def _make_conv_kernel(nb, ld, l_valid, k, with_act):
    def body(h_ref, w_ref, *rest):
        if with_act:
            a_ref, b_ref, y_ref, s_ref, ss_ref = rest
        else:
            y_ref, s_ref, ss_ref = rest
        j = pl.program_id(1)

        @pl.when(j == 0)
        def _():
            s_ref[...] = jnp.zeros_like(s_ref)
            ss_ref[...] = jnp.zeros_like(ss_ref)

        for b in range(nb):
            h = h_ref[b]
            if with_act:
                hf = h.astype(jnp.float32)
                hf = hf * a_ref[...] + b_ref[...]
                hf = jnp.where(hf > 0, hf, NEG_SLOPE * hf)
                h = hf.astype(jnp.bfloat16)
            zrow = jnp.zeros((1, h.shape[1]), jnp.bfloat16)
            acc = jnp.dot(h, w_ref[0], preferred_element_type=jnp.float32)
            for dk in range(1, k):
                hs = jnp.concatenate([h[dk:]] + [zrow] * dk, axis=0)
                acc = acc + jnp.dot(hs, w_ref[dk],
                                    preferred_element_type=jnp.float32)
            y_ref[b] = acc.astype(jnp.bfloat16)
            tail = acc[l_valid:ld]
            s_ref[0] += (jnp.sum(acc, axis=0, keepdims=True)
                         - jnp.sum(tail, axis=0, keepdims=True))
            ss_ref[0] += (jnp.sum(acc * acc, axis=0, keepdims=True)
                          - jnp.sum(tail * tail, axis=0, keepdims=True))

    return body


def _make_epilogue_kernel(nb, crop, l2):
    def body(o2_ref, res_ref, a2_ref, b2_ref, ad_ref, bd_ref, out_ref):
        for b in range(nb):
            h = res_ref[b].astype(jnp.float32)
            zrow = jnp.zeros((1, h.shape[1]), jnp.float32)
            r = jnp.concatenate([h[crop:]] + [zrow] * crop, axis=0)
            y = (o2_ref[b].astype(jnp.float32) * a2_ref[...] + b2_ref[...]
                 + r * ad_ref[...] + bd_ref[...])
            y = jnp.where(y > 0, y, NEG_SLOPE * y)
            out_ref[b] = jnp.transpose(y)

    return body


def _bn_affine(s2, ss2, count, gamma, beta):
    s = jnp.sum(s2, axis=(0, 1))
    ss = jnp.sum(ss2, axis=(0, 1))
    mean = s / count
    var = jnp.maximum(ss / count - mean * mean, 0.0)
    a = gamma * jax.lax.rsqrt(var + EPS)
    b = beta - mean * a
    return a, b


def _stats_specs(F):
    out_shape = (jax.ShapeDtypeStruct((2, 1, F), jnp.float32),
                 jax.ShapeDtypeStruct((2, 1, F), jnp.float32))
    out_specs = (pl.BlockSpec((1, 1, F), lambda i, j: (i, 0, 0)),
                 pl.BlockSpec((1, 1, F), lambda i, j: (i, 0, 0)))
    return out_shape, out_specs


def _cparams():
    return pltpu.CompilerParams(
        dimension_semantics=("parallel", "arbitrary"),
        vmem_limit_bytes=VMEM_LIMIT)


def kernel(x, wd, bd, gd, betad, w1, b1, g1, beta1, w2, b2, g2, beta2):
    N, Cin, L = x.shape
    F = wd.shape[0]
    pool = wd.shape[2]
    k = w1.shape[2]
    Ld = L // pool
    L1 = Ld - (k - 1)
    L2 = L1 - (k - 1)
    Kd = Cin * pool
    f32, bf16 = jnp.float32, jnp.bfloat16

    wdm = wd.transpose(2, 1, 0).reshape(Kd, F).astype(bf16)

    stats_shape, stats_specs = _stats_specs(F)
    NB = 8
    J2 = N // NB // 2

    h3, s_h2, ss_h2 = pl.pallas_call(
        _make_expand_kernel(NB, Ld, pool),
        grid=(2, J2),
        in_specs=[pl.BlockSpec((NB, Cin, L), lambda i, j: (i * J2 + j, 0, 0)),
                  pl.BlockSpec((Kd, F), lambda i, j: (0, 0))],
        out_specs=(pl.BlockSpec((NB, Ld, F), lambda i, j: (i * J2 + j, 0, 0)),)
        + stats_specs,
        out_shape=(jax.ShapeDtypeStruct((N, Ld, F), bf16),) + stats_shape,
        compiler_params=_cparams(),
    )(x, wdm)
    a_d, b_d = _bn_affine(s_h2, ss_h2, N * Ld, gd, betad)
    conv_in = [pl.BlockSpec((NB, Ld, F), lambda i, j: (i * J2 + j, 0, 0)),
               pl.BlockSpec((k, F, F), lambda i, j: (0, 0, 0))]
    row_spec = pl.BlockSpec((1, F), lambda i, j: (0, 0))
    conv_out = (pl.BlockSpec((NB, Ld, F), lambda i, j: (i * J2 + j, 0, 0)),) \
        + stats_specs
    conv_shape = (jax.ShapeDtypeStruct((N, Ld, F), bf16),) + stats_shape

    w1f = (w1.transpose(2, 1, 0) * a_d[None, :, None]).astype(bf16)
    o1, s_12, ss_12 = pl.pallas_call(
        _make_conv_kernel(NB, Ld, L1, k, with_act=False),
        grid=(2, J2),
        in_specs=conv_in,
        out_specs=conv_out,
        out_shape=conv_shape,
        compiler_params=_cparams(),
    )(h3, w1f)
    a_1, b_1 = _bn_affine(s_12, ss_12, N * L1, g1, beta1)

    w2f = w2.transpose(2, 1, 0).astype(bf16)
    o2, s_22, ss_22 = pl.pallas_call(
        _make_conv_kernel(NB, Ld, L2, k, with_act=True),
        grid=(2, J2),
        in_specs=conv_in + [row_spec, row_spec],
        out_specs=conv_out,
        out_shape=conv_shape,
        compiler_params=_cparams(),
    )(o1, w2f, a_1.reshape(1, F), b_1.reshape(1, F))
    a_2, b_2 = _bn_affine(s_22, ss_22, N * L2, g2, beta2)

    crop = (Ld - L2) // 2
    row = lambda v: v.reshape(1, F)
    out = pl.pallas_call(
        _make_epilogue_kernel(NB, crop, L2),
        grid=(2, J2),
        in_specs=[pl.BlockSpec((NB, Ld, F), lambda i, j: (i * J2 + j, 0, 0)),
                  pl.BlockSpec((NB, Ld, F), lambda i, j: (i * J2 + j, 0, 0)),
                  row_spec, row_spec, row_spec, row_spec],
        out_specs=pl.BlockSpec((NB, F, Ld), lambda i, j: (i * J2 + j, 0, 0)),
        out_shape=jax.ShapeDtypeStruct((N, F, Ld), f32),
        compiler_params=_cparams(),
    )(o2, h3, row(a_2), row(b_2), row(a_d), row(b_d))
    return out[:, :, :L2]
